# baseline (speedup 1.0000x reference)
import jax
import jax.numpy as jnp
from jax.experimental import pallas as pl
from jax.experimental.pallas import tpu as pltpu

N = 4096
RA = 128
RM = 128
RL = 2048
bf16 = jnp.bfloat16


def _attn_body(A_ref, x_ref, W1_ref, Wa_ref, Wa2_ref, Wa3_ref,
               G0_ref, G1_ref, G2_ref,
               ba_ref, ba2_ref, ba3_ref, bagg_ref,
               nz_ref, M1_ref, V0_s, V1_s, V2_s, c_s):
    rt = pl.program_id(0)

    @pl.when(rt == 0)
    def _fold():
        M1_ref[...] = jnp.dot(x_ref[...], W1_ref[...],
                              preferred_element_type=jnp.float32).astype(bf16)
        V0_s[...] = jnp.dot(Wa_ref[...], G0_ref[...],
                            preferred_element_type=jnp.float32).astype(bf16)
        V1_s[...] = jnp.dot(Wa2_ref[...], G1_ref[...],
                            preferred_element_type=jnp.float32).astype(bf16)
        V2_s[...] = jnp.dot(Wa3_ref[...], G2_ref[...],
                            preferred_element_type=jnp.float32).astype(bf16)
        c_s[...] = (bagg_ref[...]
                    + jnp.dot(ba_ref[...], G0_ref[...])
                    + jnp.dot(ba2_ref[...], G1_ref[...])
                    + jnp.dot(ba3_ref[...], G2_ref[...]))

    z = (jnp.dot(A_ref[0].astype(bf16), V0_s[...],
                 preferred_element_type=jnp.float32)
         + jnp.dot(A_ref[1].astype(bf16), V1_s[...],
                   preferred_element_type=jnp.float32)
         + jnp.dot(A_ref[2].astype(bf16), V2_s[...],
                   preferred_element_type=jnp.float32)
         + c_s[...])
    z = z - jnp.max(z, axis=1, keepdims=True)
    e = jnp.exp(z)
    nz_ref[...] = e / jnp.sum(e, axis=1, keepdims=True)


def _mega_body(A_ref, nzT_ref, M1_ref, Wg_ref, W2_ref,
               b1_ref, bg_ref, b2_ref, out_ref,
               adj_s, H1_s, X_s, rhs2_s, rhs3_s):
    p = pl.program_id(0)
    rt = pl.program_id(1)

    @pl.when(p == 0)
    def _mix_l1():
        rows = pl.ds(rt * RM, RM)
        nzT = nzT_ref[...]
        a = (A_ref[0] * nzT[0:1, :]
             + A_ref[1] * nzT[1:2, :]
             + A_ref[2] * nzT[2:3, :]).astype(bf16)
        adj_s[rows, :] = a
        h = jnp.dot(a, M1_ref[...], preferred_element_type=jnp.float32)
        H1_s[rows, :] = jnp.maximum(h + b1_ref[...], 0.0)

    @pl.when((p == 1) & (rt < N // RL))
    def _l2():
        @pl.when(rt == 0)
        def _():
            rhs2_s[...] = jnp.dot(
                H1_s[...], Wg_ref[...],
                preferred_element_type=jnp.float32).astype(bf16)
        rows = pl.ds(rt * RL, RL)
        xx = jnp.dot(adj_s[rows, :], rhs2_s[...],
                     preferred_element_type=jnp.float32)
        X_s[rows, :] = jnp.maximum(xx + bg_ref[...], 0.0)

    @pl.when((p == 2) & (rt < N // RL))
    def _l3():
        @pl.when(rt == 0)
        def _():
            rhs3_s[...] = jnp.dot(
                X_s[...], W2_ref[...],
                preferred_element_type=jnp.float32).astype(bf16)
        rows = pl.ds(rt * RL, RL)
        z = jnp.dot(adj_s[rows, :], rhs3_s[...],
                    preferred_element_type=jnp.float32) + b2_ref[...]
        z = z - jnp.max(z, axis=1, keepdims=True)
        e = jnp.exp(z)
        out_ref[rows, :] = e / jnp.sum(e, axis=1, keepdims=True)


def kernel(adj_list, x, adj_list_origin, Wa, ba, Wa2, ba2, Wa3, ba3,
           Wagg, bagg, W1, b1, Wg, bg, W2, b2):
    f32 = jnp.float32
    G0, G1, G2 = Wagg[0:30], Wagg[30:60], Wagg[60:90]
    ba_r, ba2_r, ba3_r = ba.reshape(1, 30), ba2.reshape(1, 30), ba3.reshape(1, 30)
    bagg_r = bagg.reshape(1, 3)
    b1_r, bg_r, b2_r = b1.reshape(1, 64), bg.reshape(1, 64), b2.reshape(1, 16)

    cmap = lambda rt: (0, 0)
    nz, M1 = pl.pallas_call(
        _attn_body,
        grid=(N // RA,),
        in_specs=[
            pl.BlockSpec((3, RA, N), lambda rt: (0, rt, 0)),
            pl.BlockSpec((N, 256), cmap),
            pl.BlockSpec((256, 64), cmap),
            pl.BlockSpec((N, 30), cmap),
            pl.BlockSpec((N, 30), cmap),
            pl.BlockSpec((N, 30), cmap),
            pl.BlockSpec((30, 3), cmap),
            pl.BlockSpec((30, 3), cmap),
            pl.BlockSpec((30, 3), cmap),
            pl.BlockSpec((1, 30), cmap),
            pl.BlockSpec((1, 30), cmap),
            pl.BlockSpec((1, 30), cmap),
            pl.BlockSpec((1, 3), cmap),
        ],
        out_specs=(
            pl.BlockSpec((RA, 3), lambda rt: (rt, 0)),
            pl.BlockSpec((N, 64), cmap),
        ),
        out_shape=(
            jax.ShapeDtypeStruct((N, 3), f32),
            jax.ShapeDtypeStruct((N, 64), bf16),
        ),
        scratch_shapes=[
            pltpu.VMEM((N, 3), bf16),
            pltpu.VMEM((N, 3), bf16),
            pltpu.VMEM((N, 3), bf16),
            pltpu.VMEM((1, 3), f32),
        ],
        compiler_params=pltpu.CompilerParams(
            dimension_semantics=("arbitrary",)),
    )(adj_list, x, W1, Wa, Wa2, Wa3, G0, G1, G2, ba_r, ba2_r, ba3_r, bagg_r)

    nzT = nz.T

    RT = N // RM
    out = pl.pallas_call(
        _mega_body,
        grid=(3, RT),
        in_specs=[
            pl.BlockSpec((3, RM, N),
                         lambda p, rt: (0, jnp.where(p == 0, rt, RT - 1), 0)),
            pl.BlockSpec((3, N), lambda p, rt: (0, 0)),
            pl.BlockSpec((N, 64), lambda p, rt: (0, 0)),
            pl.BlockSpec((64, 64), lambda p, rt: (0, 0)),
            pl.BlockSpec((64, 16), lambda p, rt: (0, 0)),
            pl.BlockSpec((1, 64), lambda p, rt: (0, 0)),
            pl.BlockSpec((1, 64), lambda p, rt: (0, 0)),
            pl.BlockSpec((1, 16), lambda p, rt: (0, 0)),
        ],
        out_specs=pl.BlockSpec((N, 16), lambda p, rt: (0, 0)),
        out_shape=jax.ShapeDtypeStruct((N, 16), f32),
        scratch_shapes=[
            pltpu.VMEM((N, N), bf16),
            pltpu.VMEM((N, 64), f32),
            pltpu.VMEM((N, 64), f32),
            pltpu.VMEM((N, 64), bf16),
            pltpu.VMEM((N, 16), bf16),
        ],
        compiler_params=pltpu.CompilerParams(
            dimension_semantics=("arbitrary", "arbitrary")),
    )(adj_list, nzT, M1, Wg, W2, b1_r, bg_r, b2_r)

    return out, nz


# speedup vs baseline: 1.0324x; 1.0324x over previous
import jax
import jax.numpy as jnp
from jax.experimental import pallas as pl
from jax.experimental.pallas import tpu as pltpu

N = 4096
RA = 256
RM = 128
RL = 2048
bf16 = jnp.bfloat16


def _attn_body(A_ref, x_ref, W1_ref, Wa_ref, Wa2_ref, Wa3_ref,
               G0_ref, G1_ref, G2_ref,
               ba_ref, ba2_ref, ba3_ref, bagg_ref,
               nz_ref, M1_ref, V0_s, V1_s, V2_s, c_s):
    rt = pl.program_id(0)

    @pl.when(rt == 0)
    def _fold():
        M1_ref[...] = jnp.dot(x_ref[...], W1_ref[...],
                              preferred_element_type=jnp.float32).astype(bf16)
        V0_s[...] = jnp.dot(Wa_ref[...], G0_ref[...],
                            preferred_element_type=jnp.float32).astype(bf16)
        V1_s[...] = jnp.dot(Wa2_ref[...], G1_ref[...],
                            preferred_element_type=jnp.float32).astype(bf16)
        V2_s[...] = jnp.dot(Wa3_ref[...], G2_ref[...],
                            preferred_element_type=jnp.float32).astype(bf16)
        c_s[...] = (bagg_ref[...]
                    + jnp.dot(ba_ref[...], G0_ref[...])
                    + jnp.dot(ba2_ref[...], G1_ref[...])
                    + jnp.dot(ba3_ref[...], G2_ref[...]))

    z = (jnp.dot(A_ref[0].astype(bf16), V0_s[...],
                 preferred_element_type=jnp.float32)
         + jnp.dot(A_ref[1].astype(bf16), V1_s[...],
                   preferred_element_type=jnp.float32)
         + jnp.dot(A_ref[2].astype(bf16), V2_s[...],
                   preferred_element_type=jnp.float32)
         + c_s[...])
    z = z - jnp.max(z, axis=1, keepdims=True)
    e = jnp.exp(z)
    nz_ref[...] = e / jnp.sum(e, axis=1, keepdims=True)


def _mega_body(A_ref, nzT_ref, M1_ref, Wg_ref, W2_ref,
               b1_ref, bg_ref, b2_ref, out_ref,
               adj_s, H1_s, X_s, rhs2_s, rhs3_s):
    p = pl.program_id(0)
    rt = pl.program_id(1)

    @pl.when(p == 0)
    def _mix_l1():
        rows = pl.ds(rt * RM, RM)
        nzT = nzT_ref[...]
        a = (A_ref[0] * nzT[0:1, :]
             + A_ref[1] * nzT[1:2, :]
             + A_ref[2] * nzT[2:3, :]).astype(bf16)
        adj_s[rows, :] = a
        h = jnp.dot(a, M1_ref[...], preferred_element_type=jnp.float32)
        H1_s[rows, :] = jnp.maximum(h + b1_ref[...], 0.0)

    @pl.when((p == 1) & (rt < N // RL))
    def _l2():
        @pl.when(rt == 0)
        def _():
            rhs2_s[...] = jnp.dot(
                H1_s[...], Wg_ref[...],
                preferred_element_type=jnp.float32).astype(bf16)
        rows = pl.ds(rt * RL, RL)
        xx = jnp.dot(adj_s[rows, :], rhs2_s[...],
                     preferred_element_type=jnp.float32)
        X_s[rows, :] = jnp.maximum(xx + bg_ref[...], 0.0)

    @pl.when((p == 2) & (rt < N // RL))
    def _l3():
        @pl.when(rt == 0)
        def _():
            rhs3_s[...] = jnp.dot(
                X_s[...], W2_ref[...],
                preferred_element_type=jnp.float32).astype(bf16)
        rows = pl.ds(rt * RL, RL)
        z = jnp.dot(adj_s[rows, :], rhs3_s[...],
                    preferred_element_type=jnp.float32) + b2_ref[...]
        z = z - jnp.max(z, axis=1, keepdims=True)
        e = jnp.exp(z)
        out_ref[rows, :] = e / jnp.sum(e, axis=1, keepdims=True)


def kernel(adj_list, x, adj_list_origin, Wa, ba, Wa2, ba2, Wa3, ba3,
           Wagg, bagg, W1, b1, Wg, bg, W2, b2):
    f32 = jnp.float32
    G0, G1, G2 = Wagg[0:30], Wagg[30:60], Wagg[60:90]
    ba_r, ba2_r, ba3_r = ba.reshape(1, 30), ba2.reshape(1, 30), ba3.reshape(1, 30)
    bagg_r = bagg.reshape(1, 3)
    b1_r, bg_r, b2_r = b1.reshape(1, 64), bg.reshape(1, 64), b2.reshape(1, 16)

    cmap = lambda rt: (0, 0)
    nz, M1 = pl.pallas_call(
        _attn_body,
        grid=(N // RA,),
        in_specs=[
            pl.BlockSpec((3, RA, N), lambda rt: (0, rt, 0)),
            pl.BlockSpec((N, 256), cmap),
            pl.BlockSpec((256, 64), cmap),
            pl.BlockSpec((N, 30), cmap),
            pl.BlockSpec((N, 30), cmap),
            pl.BlockSpec((N, 30), cmap),
            pl.BlockSpec((30, 3), cmap),
            pl.BlockSpec((30, 3), cmap),
            pl.BlockSpec((30, 3), cmap),
            pl.BlockSpec((1, 30), cmap),
            pl.BlockSpec((1, 30), cmap),
            pl.BlockSpec((1, 30), cmap),
            pl.BlockSpec((1, 3), cmap),
        ],
        out_specs=(
            pl.BlockSpec((RA, 3), lambda rt: (rt, 0)),
            pl.BlockSpec((N, 64), cmap),
        ),
        out_shape=(
            jax.ShapeDtypeStruct((N, 3), f32),
            jax.ShapeDtypeStruct((N, 64), bf16),
        ),
        scratch_shapes=[
            pltpu.VMEM((N, 3), bf16),
            pltpu.VMEM((N, 3), bf16),
            pltpu.VMEM((N, 3), bf16),
            pltpu.VMEM((1, 3), f32),
        ],
        compiler_params=pltpu.CompilerParams(
            dimension_semantics=("arbitrary",)),
    )(adj_list, x, W1, Wa, Wa2, Wa3, G0, G1, G2, ba_r, ba2_r, ba3_r, bagg_r)

    nzT = nz.T

    RT = N // RM
    out = pl.pallas_call(
        _mega_body,
        grid=(3, RT),
        in_specs=[
            pl.BlockSpec((3, RM, N),
                         lambda p, rt: (0, jnp.where(p == 0, rt, RT - 1), 0)),
            pl.BlockSpec((3, N), lambda p, rt: (0, 0)),
            pl.BlockSpec((N, 64), lambda p, rt: (0, 0)),
            pl.BlockSpec((64, 64), lambda p, rt: (0, 0)),
            pl.BlockSpec((64, 16), lambda p, rt: (0, 0)),
            pl.BlockSpec((1, 64), lambda p, rt: (0, 0)),
            pl.BlockSpec((1, 64), lambda p, rt: (0, 0)),
            pl.BlockSpec((1, 16), lambda p, rt: (0, 0)),
        ],
        out_specs=pl.BlockSpec((N, 16), lambda p, rt: (0, 0)),
        out_shape=jax.ShapeDtypeStruct((N, 16), f32),
        scratch_shapes=[
            pltpu.VMEM((N, N), bf16),
            pltpu.VMEM((N, 64), f32),
            pltpu.VMEM((N, 64), f32),
            pltpu.VMEM((N, 64), bf16),
            pltpu.VMEM((N, 16), bf16),
        ],
        compiler_params=pltpu.CompilerParams(
            dimension_semantics=("arbitrary", "arbitrary")),
    )(adj_list, nzT, M1, Wg, W2, b1_r, bg_r, b2_r)

    return out, nz
